# baseline (speedup 1.0000x reference)
import functools

import jax
import jax.numpy as jnp
from jax import lax
from jax.experimental import pallas as pl
from jax.experimental.pallas import tpu as pltpu
from jax.experimental.pallas import tpu_sc as plsc

B = 1024
D = 128
E = 100000
EP = 100352
ET = 1024
T = EP // ET
CHUNK = 128
CPT = ET // CHUNK
NCH = EP // CHUNK
BT = 1024
R = B // BT
K = 8
NCLS = 10
NEG = float(jnp.finfo(jnp.float32).min)
IMAX = 2**31 - 1


def _extract_topk(wv, wi, k):
    vs, is_ = [], []
    for _ in range(k):
        m = jnp.max(wv, axis=1, keepdims=True)
        cand = jnp.where(wv == m, wi, IMAX)
        a = jnp.min(cand, axis=1, keepdims=True)
        vs.append(m)
        is_.append(a)
        wv = jnp.where(wi == a, NEG, wv)
    return jnp.concatenate(vs, axis=1), jnp.concatenate(is_, axis=1)


def _k1_body(x_ref, ev_ref, scores_ref, ids_ref, cmax_ref):
    r = pl.program_id(0)
    t = pl.program_id(1)
    x = x_ref[...]
    ev = ev_ref[...]
    s = lax.dot_general(
        x, ev, (((1,), (1,)), ((), ())),
        preferred_element_type=jnp.float32)

    def finish(sv):
        scores_ref[...] = sv
        cm = jnp.max(sv.reshape(BT, CPT, CHUNK), axis=2)
        cmax_ref[pl.ds(t * CPT, CPT), :] = cm.T

    @pl.when(t < T - 1)
    def _():
        finish(s)

    @pl.when(t == T - 1)
    def _():
        col = (T - 1) * ET + lax.broadcasted_iota(jnp.int32, (BT, ET), 1)
        finish(jnp.where(col < E, s, NEG))
        wv = cmax_ref[...]
        wi = lax.broadcasted_iota(jnp.int32, (NCH, BT), 0)
        ni = []
        for _ in range(K):
            m = jnp.max(wv, axis=0, keepdims=True)
            cand = jnp.where(wv == m, wi, IMAX)
            a = jnp.min(cand, axis=0, keepdims=True)
            ni.append(a)
            wv = jnp.where(wi == a, NEG, wv)
        ni = jnp.concatenate(ni, axis=0)
        grow = r * BT + lax.broadcasted_iota(jnp.int32, (K, BT), 1)
        ids_ref[...] = ni + grow * NCH


def _k1(graph_feature, ev_pad):
    return pl.pallas_call(
        _k1_body,
        grid=(R, T),
        in_specs=[
            pl.BlockSpec((BT, D), lambda r, t: (r, 0)),
            pl.BlockSpec((ET, D), lambda r, t: (t, 0)),
        ],
        out_specs=[
            pl.BlockSpec((BT, ET), lambda r, t: (r, t)),
            pl.BlockSpec((K, BT), lambda r, t: (0, r)),
        ],
        out_shape=[
            jax.ShapeDtypeStruct((B, EP), jnp.float32),
            jax.ShapeDtypeStruct((K, B), jnp.int32),
        ],
        scratch_shapes=[
            pltpu.VMEM((NCH, BT), jnp.float32),
        ],
    )(graph_feature, ev_pad)


def _sc_gather_rows(table, ids, n_rows, row_w, dtype):
    nw = 32
    per = ids.shape[0] // nw
    j_steps = per // 128
    mesh = plsc.VectorSubcoreMesh(core_axis_name="c", subcore_axis_name="s")

    @functools.partial(
        pl.kernel,
        out_type=jax.ShapeDtypeStruct((ids.shape[0], row_w), dtype),
        mesh=mesh,
        scratch_types=[
            pltpu.VMEM((j_steps, 128), jnp.int32),
            pltpu.VMEM((j_steps, 128, row_w), dtype),
            pltpu.SemaphoreType.DMA,
        ],
    )
    def k(table_hbm, ids_hbm, out_hbm, idx_v, rows_v, sem):
        wid = lax.axis_index("s") * 2 + lax.axis_index("c")
        base = wid * per
        for j in range(j_steps):
            pltpu.sync_copy(ids_hbm.at[pl.ds(base + j * 128, 128)], idx_v.at[j])
            pltpu.async_copy(table_hbm.at[idx_v.at[j]], rows_v.at[j], sem).wait()
            pltpu.sync_copy(rows_v.at[j], out_hbm.at[pl.ds(base + j * 128, 128)])

    return k(table, ids)


def _sc_gather_ev_and_labels(ev, combo, ids, chunk_ids):
    nw = 32
    n = ids.shape[0]
    per = n // nw
    j_steps = per // 128
    mesh = plsc.VectorSubcoreMesh(core_axis_name="c", subcore_axis_name="s")

    @functools.partial(
        pl.kernel,
        out_type=(
            jax.ShapeDtypeStruct((n, D), jnp.float32),
            jax.ShapeDtypeStruct((n, CHUNK), jnp.int32),
        ),
        mesh=mesh,
        scratch_types=[
            pltpu.VMEM((j_steps, 128), jnp.int32),
            pltpu.VMEM((j_steps, 128), jnp.int32),
            pltpu.VMEM((j_steps, 128, D), jnp.float32),
            pltpu.VMEM((j_steps, 128, CHUNK), jnp.int32),
            pltpu.SemaphoreType.DMA,
            pltpu.SemaphoreType.DMA,
        ],
    )
    def k(ev_hbm, lab_hbm, ids_hbm, cids_hbm, kev_hbm, krow_hbm, idx_v,
          cidx_v, ev_v, lab_v, sem1, sem2):
        wid = lax.axis_index("s") * 2 + lax.axis_index("c")
        base = wid * per
        for j in range(j_steps):
            pltpu.sync_copy(ids_hbm.at[pl.ds(base + j * 128, 128)], idx_v.at[j])
            pltpu.sync_copy(cids_hbm.at[pl.ds(base + j * 128, 128)],
                            cidx_v.at[j])
            c1 = pltpu.async_copy(ev_hbm.at[idx_v.at[j]], ev_v.at[j], sem1)
            c2 = pltpu.async_copy(lab_hbm.at[cidx_v.at[j]], lab_v.at[j], sem2)
            c1.wait()
            c2.wait()
            pltpu.sync_copy(ev_v.at[j], kev_hbm.at[pl.ds(base + j * 128, 128)])
            pltpu.sync_copy(lab_v.at[j],
                            krow_hbm.at[pl.ds(base + j * 128, 128)])

    return k(ev, combo, ids, chunk_ids)


def _k3_body(cands_ref, ids_ref, kind_ref, kchunk_ref, kmod_ref):
    row = lax.broadcasted_iota(jnp.int32, (K, B), 1)
    chunk = ids_ref[...] - row * NCH
    wi = (chunk.reshape(K, B, 1) * CHUNK
          + lax.broadcasted_iota(jnp.int32, (K, B, CHUNK), 2))
    wv = cands_ref[...]
    ni = []
    for _ in range(K):
        m = jnp.max(jnp.max(wv, axis=2), axis=0, keepdims=True)
        cand = jnp.where(wv == m[:, :, None], wi, IMAX)
        a = jnp.min(jnp.min(cand, axis=2), axis=0, keepdims=True)
        ni.append(a)
        wv = jnp.where(wi == a[:, :, None], NEG, wv)
    kv = jnp.concatenate(ni, axis=0)
    kind_ref[...] = kv
    kchunk_ref[...] = kv >> 7
    kmod_ref[...] = kv & (CHUNK - 1)


def _k3(cands, cids):
    return pl.pallas_call(
        _k3_body,
        out_shape=[
            jax.ShapeDtypeStruct((K, B), jnp.int32),
            jax.ShapeDtypeStruct((K, B), jnp.int32),
            jax.ShapeDtypeStruct((K, B), jnp.int32),
        ],
    )(cands, cids)


def _k5_body(gf_ref, kev_ref, krow_ref, kmod_ref, le0_ref, le1_ref, w1_ref,
             b1_ref, w2_ref, b2_ref, out_ref):
    dot = functools.partial(
        lax.dot_general,
        dimension_numbers=(((1,), (0,)), ((), ())),
        preferred_element_type=jnp.float32,
        precision=lax.Precision.HIGHEST)
    w1a = w1_ref[0:D, :]
    w1b = w1_ref[D:2 * D, :]
    w1c = w1_ref[2 * D:3 * D, :]
    t0 = dot(le0_ref[...], w1b)
    t1 = dot(le1_ref[...], w1c)
    cls = lax.broadcasted_iota(jnp.int32, (B * K, 16), 1)
    lane = lax.broadcasted_iota(jnp.int32, (B * K, CHUNK), 1)
    code = jnp.sum(jnp.where(kmod_ref[...] == lane, krow_ref[...], 0),
                   axis=1, keepdims=True)
    oh0 = ((code & 15) == cls).astype(jnp.float32)
    oh1 = ((code >> 4) == cls).astype(jnp.float32)
    h = dot(kev_ref[...], w1a) + dot(oh0, t0) + dot(oh1, t1) + b1_ref[...]
    h = jnp.maximum(h, 0.0)
    pooled = jnp.sum(h.reshape(K, B, D), axis=0)
    out2 = dot(pooled, w2_ref[...]) + b2_ref[...]
    out_ref[...] = jnp.concatenate([gf_ref[...], out2], axis=1)


def _k5(gf, kev, krow, kmod, le0p, le1p, w1, b1, w2, b2):
    return pl.pallas_call(
        _k5_body,
        out_shape=jax.ShapeDtypeStruct((B, 2 * D), jnp.float32),
    )(gf, kev, krow, kmod, le0p, le1p, w1, b1, w2, b2)


def kernel(graph_feature, evidence_emb, label_emb_0, label_emb_1, W1, b1, W2,
           b2, emb_label):
    lab = emb_label.astype(jnp.int32)
    combo = jnp.pad(lab[0] + 16 * lab[1], (0, EP - E)).reshape(NCH, CHUNK)
    le0p = jnp.pad(label_emb_0, ((0, 16 - NCLS), (0, 0)))
    le1p = jnp.pad(label_emb_1, ((0, 16 - NCLS), (0, 0)))

    scores, cids = _k1(graph_feature, evidence_emb)
    cands = _sc_gather_rows(
        scores.reshape(B * NCH, CHUNK), cids.reshape(K * B),
        B * NCH, CHUNK, jnp.float32)
    kind, kchunk, kmod = _k3(cands.reshape(K, B, CHUNK), cids)
    kev, krow = _sc_gather_ev_and_labels(evidence_emb, combo,
                                         kind.reshape(K * B),
                                         kchunk.reshape(K * B))
    return _k5(graph_feature, kev, krow, kmod.reshape(K * B, 1), le0p, le1p,
               W1, b1.reshape(1, D), W2, b2.reshape(1, D))


# speedup vs baseline: 1.0311x; 1.0311x over previous
import functools

import jax
import jax.numpy as jnp
from jax import lax
from jax.experimental import pallas as pl
from jax.experimental.pallas import tpu as pltpu
from jax.experimental.pallas import tpu_sc as plsc

B = 1024
D = 128
E = 100000
EP = 100352
ET = 2048
T = EP // ET
CHUNK = 128
CPT = ET // CHUNK
NCH = EP // CHUNK
BT = 1024
R = B // BT
K = 8
NCLS = 10
NEG = float(jnp.finfo(jnp.float32).min)
IMAX = 2**31 - 1


def _extract_topk(wv, wi, k):
    vs, is_ = [], []
    for _ in range(k):
        m = jnp.max(wv, axis=1, keepdims=True)
        cand = jnp.where(wv == m, wi, IMAX)
        a = jnp.min(cand, axis=1, keepdims=True)
        vs.append(m)
        is_.append(a)
        wv = jnp.where(wi == a, NEG, wv)
    return jnp.concatenate(vs, axis=1), jnp.concatenate(is_, axis=1)


def _k1_body(x_ref, ev_ref, scores_ref, ids_ref, cmax_ref):
    r = pl.program_id(0)
    t = pl.program_id(1)
    x = x_ref[...]
    ev = ev_ref[...]
    s = lax.dot_general(
        x, ev, (((1,), (1,)), ((), ())),
        preferred_element_type=jnp.float32)

    def finish(sv):
        scores_ref[...] = sv
        cm = jnp.max(sv.reshape(BT, CPT, CHUNK), axis=2)
        cmax_ref[pl.ds(t * CPT, CPT), :] = cm.T

    @pl.when(t < T - 1)
    def _():
        finish(s)

    @pl.when(t == T - 1)
    def _():
        col = (T - 1) * ET + lax.broadcasted_iota(jnp.int32, (BT, ET), 1)
        finish(jnp.where(col < E, s, NEG))
        wv = cmax_ref[...]
        wi = lax.broadcasted_iota(jnp.int32, (NCH, BT), 0)
        ni = []
        for _ in range(K):
            m = jnp.max(wv, axis=0, keepdims=True)
            cand = jnp.where(wv == m, wi, IMAX)
            a = jnp.min(cand, axis=0, keepdims=True)
            ni.append(a)
            wv = jnp.where(wi == a, NEG, wv)
        ni = jnp.concatenate(ni, axis=0)
        grow = r * BT + lax.broadcasted_iota(jnp.int32, (K, BT), 1)
        ids_ref[...] = ni + grow * NCH


def _k1(graph_feature, ev_pad):
    return pl.pallas_call(
        _k1_body,
        grid=(R, T),
        in_specs=[
            pl.BlockSpec((BT, D), lambda r, t: (r, 0)),
            pl.BlockSpec((ET, D), lambda r, t: (t, 0)),
        ],
        out_specs=[
            pl.BlockSpec((BT, ET), lambda r, t: (r, t)),
            pl.BlockSpec((K, BT), lambda r, t: (0, r)),
        ],
        out_shape=[
            jax.ShapeDtypeStruct((B, EP), jnp.float32),
            jax.ShapeDtypeStruct((K, B), jnp.int32),
        ],
        scratch_shapes=[
            pltpu.VMEM((NCH, BT), jnp.float32),
        ],
    )(graph_feature, ev_pad)


def _sc_gather_rows(table, ids, n_rows, row_w, dtype):
    nw = 32
    per = ids.shape[0] // nw
    j_steps = per // 128
    mesh = plsc.VectorSubcoreMesh(core_axis_name="c", subcore_axis_name="s")

    @functools.partial(
        pl.kernel,
        out_type=jax.ShapeDtypeStruct((ids.shape[0], row_w), dtype),
        mesh=mesh,
        scratch_types=[
            pltpu.VMEM((j_steps, 128), jnp.int32),
            pltpu.VMEM((j_steps, 128, row_w), dtype),
            pltpu.SemaphoreType.DMA,
        ],
    )
    def k(table_hbm, ids_hbm, out_hbm, idx_v, rows_v, sem):
        wid = lax.axis_index("s") * 2 + lax.axis_index("c")
        base = wid * per
        for j in range(j_steps):
            pltpu.sync_copy(ids_hbm.at[pl.ds(base + j * 128, 128)], idx_v.at[j])
            pltpu.async_copy(table_hbm.at[idx_v.at[j]], rows_v.at[j], sem).wait()
            pltpu.sync_copy(rows_v.at[j], out_hbm.at[pl.ds(base + j * 128, 128)])

    return k(table, ids)


def _sc_gather_ev_and_labels(ev, combo, ids, chunk_ids):
    nw = 32
    n = ids.shape[0]
    per = n // nw
    j_steps = per // 128
    mesh = plsc.VectorSubcoreMesh(core_axis_name="c", subcore_axis_name="s")

    @functools.partial(
        pl.kernel,
        out_type=(
            jax.ShapeDtypeStruct((n, D), jnp.float32),
            jax.ShapeDtypeStruct((n, CHUNK), jnp.int32),
        ),
        mesh=mesh,
        scratch_types=[
            pltpu.VMEM((j_steps, 128), jnp.int32),
            pltpu.VMEM((j_steps, 128), jnp.int32),
            pltpu.VMEM((j_steps, 128, D), jnp.float32),
            pltpu.VMEM((j_steps, 128, CHUNK), jnp.int32),
            pltpu.SemaphoreType.DMA,
            pltpu.SemaphoreType.DMA,
        ],
    )
    def k(ev_hbm, lab_hbm, ids_hbm, cids_hbm, kev_hbm, krow_hbm, idx_v,
          cidx_v, ev_v, lab_v, sem1, sem2):
        wid = lax.axis_index("s") * 2 + lax.axis_index("c")
        base = wid * per
        for j in range(j_steps):
            pltpu.sync_copy(ids_hbm.at[pl.ds(base + j * 128, 128)], idx_v.at[j])
            pltpu.sync_copy(cids_hbm.at[pl.ds(base + j * 128, 128)],
                            cidx_v.at[j])
            c1 = pltpu.async_copy(ev_hbm.at[idx_v.at[j]], ev_v.at[j], sem1)
            c2 = pltpu.async_copy(lab_hbm.at[cidx_v.at[j]], lab_v.at[j], sem2)
            c1.wait()
            c2.wait()
            pltpu.sync_copy(ev_v.at[j], kev_hbm.at[pl.ds(base + j * 128, 128)])
            pltpu.sync_copy(lab_v.at[j],
                            krow_hbm.at[pl.ds(base + j * 128, 128)])

    return k(ev, combo, ids, chunk_ids)


def _k3_body(cands_ref, ids_ref, kind_ref, kchunk_ref, kmod_ref):
    row = lax.broadcasted_iota(jnp.int32, (K, B), 1)
    chunk = ids_ref[...] - row * NCH
    wi = (chunk.reshape(K, B, 1) * CHUNK
          + lax.broadcasted_iota(jnp.int32, (K, B, CHUNK), 2))
    wv = cands_ref[...]
    ni = []
    for _ in range(K):
        m = jnp.max(jnp.max(wv, axis=2), axis=0, keepdims=True)
        cand = jnp.where(wv == m[:, :, None], wi, IMAX)
        a = jnp.min(jnp.min(cand, axis=2), axis=0, keepdims=True)
        ni.append(a)
        wv = jnp.where(wi == a[:, :, None], NEG, wv)
    kv = jnp.concatenate(ni, axis=0)
    kind_ref[...] = kv
    kchunk_ref[...] = kv >> 7
    kmod_ref[...] = kv & (CHUNK - 1)


def _k3(cands, cids):
    return pl.pallas_call(
        _k3_body,
        out_shape=[
            jax.ShapeDtypeStruct((K, B), jnp.int32),
            jax.ShapeDtypeStruct((K, B), jnp.int32),
            jax.ShapeDtypeStruct((K, B), jnp.int32),
        ],
    )(cands, cids)


def _k5_body(gf_ref, kev_ref, krow_ref, kmod_ref, le0_ref, le1_ref, w1_ref,
             b1_ref, w2_ref, b2_ref, out_ref):
    dot = functools.partial(
        lax.dot_general,
        dimension_numbers=(((1,), (0,)), ((), ())),
        preferred_element_type=jnp.float32,
        precision=lax.Precision.HIGHEST)
    w1a = w1_ref[0:D, :]
    w1b = w1_ref[D:2 * D, :]
    w1c = w1_ref[2 * D:3 * D, :]
    t0 = dot(le0_ref[...], w1b)
    t1 = dot(le1_ref[...], w1c)
    cls = lax.broadcasted_iota(jnp.int32, (B * K, 16), 1)
    lane = lax.broadcasted_iota(jnp.int32, (B * K, CHUNK), 1)
    code = jnp.sum(jnp.where(kmod_ref[...] == lane, krow_ref[...], 0),
                   axis=1, keepdims=True)
    oh0 = ((code & 15) == cls).astype(jnp.float32)
    oh1 = ((code >> 4) == cls).astype(jnp.float32)
    h = dot(kev_ref[...], w1a) + dot(oh0, t0) + dot(oh1, t1) + b1_ref[...]
    h = jnp.maximum(h, 0.0)
    pooled = jnp.sum(h.reshape(K, B, D), axis=0)
    out2 = dot(pooled, w2_ref[...]) + b2_ref[...]
    out_ref[...] = jnp.concatenate([gf_ref[...], out2], axis=1)


def _k5(gf, kev, krow, kmod, le0p, le1p, w1, b1, w2, b2):
    return pl.pallas_call(
        _k5_body,
        out_shape=jax.ShapeDtypeStruct((B, 2 * D), jnp.float32),
    )(gf, kev, krow, kmod, le0p, le1p, w1, b1, w2, b2)


def kernel(graph_feature, evidence_emb, label_emb_0, label_emb_1, W1, b1, W2,
           b2, emb_label):
    lab = emb_label.astype(jnp.int32)
    combo = jnp.pad(lab[0] + 16 * lab[1], (0, EP - E)).reshape(NCH, CHUNK)
    le0p = jnp.pad(label_emb_0, ((0, 16 - NCLS), (0, 0)))
    le1p = jnp.pad(label_emb_1, ((0, 16 - NCLS), (0, 0)))

    scores, cids = _k1(graph_feature, evidence_emb)
    cands = _sc_gather_rows(
        scores.reshape(B * NCH, CHUNK), cids.reshape(K * B),
        B * NCH, CHUNK, jnp.float32)
    kind, kchunk, kmod = _k3(cands.reshape(K, B, CHUNK), cids)
    kev, krow = _sc_gather_ev_and_labels(evidence_emb, combo,
                                         kind.reshape(K * B),
                                         kchunk.reshape(K * B))
    return _k5(graph_feature, kev, krow, kmod.reshape(K * B, 1), le0p, le1p,
               W1, b1.reshape(1, D), W2, b2.reshape(1, D))
